# baseline (speedup 1.0000x reference)
import functools

import jax
import jax.numpy as jnp
from jax import lax
from jax.experimental import pallas as pl
from jax.experimental.pallas import tpu as pltpu
from jax.experimental.pallas import tpu_sc as plsc

_VOCAB = 100000
_EMBED = 64
_BATCH = 1024

_NC = 2
_NS = 16
_NW = _NC * _NS
_B_PER_W = _BATCH // _NW

_BN = 2048
_NV = pl.cdiv(_VOCAB, _BN)


def _sc_gather(table, idx):
    mesh = plsc.VectorSubcoreMesh(core_axis_name="c", subcore_axis_name="s")

    @functools.partial(
        pl.kernel,
        mesh=mesh,
        out_type=jax.ShapeDtypeStruct((_BATCH, _EMBED), jnp.float32),
        scratch_types=[
            pltpu.VMEM((_B_PER_W,), jnp.int32),
            pltpu.VMEM((_B_PER_W, _EMBED), jnp.float32),
            pltpu.SemaphoreType.DMA,
        ],
        compiler_params=pltpu.CompilerParams(use_tc_tiling_on_sc=False),
    )
    def gather_kernel(table_hbm, idx_hbm, out_hbm, idx_v, rows_v, sem):
        wid = lax.axis_index("s") * _NC + lax.axis_index("c")
        base = wid * _B_PER_W
        pltpu.sync_copy(idx_hbm.at[pl.ds(base, _B_PER_W)], idx_v)
        pltpu.async_copy(table_hbm.at[idx_v], rows_v, sem).wait()
        pltpu.sync_copy(rows_v, out_hbm.at[pl.ds(base, _B_PER_W)])

    return gather_kernel(table, idx)


def _mlp_body(e_ref, wh_ref, bh_ref, wo_ref, bo_ref, out_ref, h_s):
    @pl.when(pl.program_id(0) == 0)
    def _():
        h = jnp.dot(e_ref[...], wh_ref[...], preferred_element_type=jnp.float32)
        h_s[...] = jnp.maximum(h + bh_ref[...], 0.0)

    out = lax.dot_general(
        wo_ref[...], h_s[...], (((0,), (1,)), ((), ())),
        preferred_element_type=jnp.float32)
    ones = jnp.ones((1, _BATCH), dtype=jnp.float32)
    out_ref[...] = out + lax.dot_general(
        bo_ref[...], ones, (((0,), (0,)), ((), ())),
        preferred_element_type=jnp.float32)


def _tc_mlp(e, W_h, b_h, W_o, b_o):
    return pl.pallas_call(
        _mlp_body,
        grid=(_NV,),
        in_specs=[
            pl.BlockSpec((_BATCH, _EMBED), lambda i: (0, 0)),
            pl.BlockSpec((_EMBED, _EMBED), lambda i: (0, 0)),
            pl.BlockSpec((1, _EMBED), lambda i: (0, 0)),
            pl.BlockSpec((_EMBED, _BN), lambda i: (0, i)),
            pl.BlockSpec((1, _BN), lambda i: (0, i)),
        ],
        out_specs=pl.BlockSpec((_BN, _BATCH), lambda i: (i, 0)),
        out_shape=jax.ShapeDtypeStruct((_VOCAB, _BATCH), jnp.float32),
        scratch_shapes=[pltpu.VMEM((_BATCH, _EMBED), jnp.float32)],
        compiler_params=pltpu.CompilerParams(
            dimension_semantics=("arbitrary",),
        ),
    )(e, W_h, b_h, W_o, b_o)


@jax.jit
def kernel(x, table, W_h, b_h, W_o, b_o):
    idx = x.astype(jnp.int32)
    e = _sc_gather(table, idx)
    out_t = _tc_mlp(e, W_h, jnp.reshape(b_h, (1, _EMBED)), W_o,
                    jnp.reshape(b_o, (1, _VOCAB)))
    return jnp.transpose(out_t)


# speedup vs baseline: 1.1897x; 1.1897x over previous
import functools

import jax
import jax.numpy as jnp
from jax import lax
from jax.experimental import pallas as pl
from jax.experimental.pallas import tpu as pltpu
from jax.experimental.pallas import tpu_sc as plsc

_VOCAB = 100000
_EMBED = 64
_BATCH = 1024
_EP = 128

_NC = 2
_NS = 16
_NW = _NC * _NS
_B_PER_W = _BATCH // _NW

_BT = 8192
_NT = pl.cdiv(_VOCAB, _BT)
_BN = 2048
_NV = pl.cdiv(_VOCAB, _BN)


def _tw_body(tt_ref, wh_ref, out_ref):
    tw = lax.dot_general(
        tt_ref[...], wh_ref[...], (((0,), (0,)), ((), ())),
        preferred_element_type=jnp.float32)
    out_ref[:, : _EMBED] = tw
    out_ref[:, _EMBED :] = jnp.zeros((_BT, _EP - _EMBED), jnp.float32)


def _tc_table_transform(table_t, W_h):
    return pl.pallas_call(
        _tw_body,
        grid=(_NT,),
        in_specs=[
            pl.BlockSpec((_EMBED, _BT), lambda i: (0, i)),
            pl.BlockSpec((_EMBED, _EMBED), lambda i: (0, 0)),
        ],
        out_specs=pl.BlockSpec((_BT, _EP), lambda i: (i, 0)),
        out_shape=jax.ShapeDtypeStruct((_VOCAB, _EP), jnp.float32),
        compiler_params=pltpu.CompilerParams(
            dimension_semantics=("arbitrary",),
        ),
    )(table_t, W_h)


def _sc_gather(tw, idx):
    mesh = plsc.VectorSubcoreMesh(core_axis_name="c", subcore_axis_name="s")

    @functools.partial(
        pl.kernel,
        mesh=mesh,
        out_type=jax.ShapeDtypeStruct((_BATCH, _EP), jnp.float32),
        scratch_types=[
            pltpu.VMEM((_B_PER_W,), jnp.int32),
            pltpu.VMEM((_B_PER_W, _EP), jnp.float32),
            pltpu.SemaphoreType.DMA,
        ],
    )
    def gather_kernel(tw_hbm, idx_hbm, out_hbm, idx_v, rows_v, sem):
        wid = lax.axis_index("s") * _NC + lax.axis_index("c")
        base = wid * _B_PER_W
        pltpu.sync_copy(idx_hbm.at[pl.ds(base, _B_PER_W)], idx_v)
        pltpu.async_copy(tw_hbm.at[idx_v], rows_v, sem).wait()
        pltpu.sync_copy(rows_v, out_hbm.at[pl.ds(base, _B_PER_W)])

    return gather_kernel(tw, idx)


def _mlp_body(hp_ref, bh_ref, wo_ref, bo_ref, out_ref, h_s):
    @pl.when(pl.program_id(0) == 0)
    def _():
        h_s[...] = jnp.maximum(hp_ref[:, : _EMBED] + bh_ref[...], 0.0)

    out = lax.dot_general(
        wo_ref[...], h_s[...], (((0,), (1,)), ((), ())),
        preferred_element_type=jnp.float32)
    ones = jnp.ones((1, _BATCH), dtype=jnp.float32)
    out_ref[...] = out + lax.dot_general(
        bo_ref[...], ones, (((0,), (0,)), ((), ())),
        preferred_element_type=jnp.float32)


def _tc_mlp(hp, b_h, W_o, b_o):
    return pl.pallas_call(
        _mlp_body,
        grid=(_NV,),
        in_specs=[
            pl.BlockSpec((_BATCH, _EP), lambda i: (0, 0)),
            pl.BlockSpec((1, _EMBED), lambda i: (0, 0)),
            pl.BlockSpec((_EMBED, _BN), lambda i: (0, i)),
            pl.BlockSpec((1, _BN), lambda i: (0, i)),
        ],
        out_specs=pl.BlockSpec((_BN, _BATCH), lambda i: (i, 0)),
        out_shape=jax.ShapeDtypeStruct((_VOCAB, _BATCH), jnp.float32),
        scratch_shapes=[pltpu.VMEM((_BATCH, _EMBED), jnp.float32)],
        compiler_params=pltpu.CompilerParams(
            dimension_semantics=("arbitrary",),
        ),
    )(hp, b_h, W_o, b_o)


@jax.jit
def kernel(x, table, W_h, b_h, W_o, b_o):
    idx = x.astype(jnp.int32)
    table_t = jnp.transpose(table)
    tw = _tc_table_transform(table_t, W_h)
    hp = _sc_gather(tw, idx)
    out_t = _tc_mlp(hp, jnp.reshape(b_h, (1, _EMBED)), W_o,
                    jnp.reshape(b_o, (1, _VOCAB)))
    return jnp.transpose(out_t)


# speedup vs baseline: 1.1922x; 1.0021x over previous
import functools

import jax
import jax.numpy as jnp
from jax import lax
from jax.experimental import pallas as pl
from jax.experimental.pallas import tpu as pltpu
from jax.experimental.pallas import tpu_sc as plsc

_VOCAB = 100000
_EMBED = 64
_BATCH = 1024
_PK = 2
_RW = _PK * _EMBED
_HV = 51200

_NC = 2
_NS = 16
_NW = _NC * _NS
_B_PER_W = _BATCH // _NW
_L = 16

_BT = 5120
_NT = _HV // _BT
_BN = 2048
_NV = pl.cdiv(_VOCAB, _BN)


def _tw_body(tt_lo_ref, tt_hi_ref, wh_ref, out_ref):
    lo = lax.dot_general(
        tt_lo_ref[...], wh_ref[...], (((0,), (0,)), ((), ())),
        preferred_element_type=jnp.float32)
    hi = lax.dot_general(
        tt_hi_ref[...], wh_ref[...], (((0,), (0,)), ((), ())),
        preferred_element_type=jnp.float32)
    out_ref[...] = jnp.concatenate([lo, hi], axis=1)


def _tc_table_transform(table_t, W_h):
    return pl.pallas_call(
        _tw_body,
        grid=(_NT,),
        in_specs=[
            pl.BlockSpec((_EMBED, _BT), lambda i: (0, i)),
            pl.BlockSpec((_EMBED, _BT), lambda i: (0, i + _NT)),
            pl.BlockSpec((_EMBED, _EMBED), lambda i: (0, 0)),
        ],
        out_specs=pl.BlockSpec((_BT, _RW), lambda i: (i, 0)),
        out_shape=jax.ShapeDtypeStruct((_HV, _RW), jnp.float32),
        compiler_params=pltpu.CompilerParams(
            dimension_semantics=("arbitrary",),
        ),
    )(table_t, table_t, W_h)


def _sc_gather(tw, idxp):
    mesh = plsc.VectorSubcoreMesh(core_axis_name="c", subcore_axis_name="s")

    @functools.partial(
        pl.kernel,
        mesh=mesh,
        out_type=jax.ShapeDtypeStruct((_BATCH, _RW), jnp.float32),
        scratch_types=[
            pltpu.VMEM((_B_PER_W,), jnp.int32),
            pltpu.VMEM((_B_PER_W, _RW), jnp.float32),
            pltpu.SemaphoreType.DMA,
        ],
    )
    def gather_kernel(tw_hbm, idx_hbm, out_hbm, idx_v, rows_v, sem):
        wid = lax.axis_index("s") * _NC + lax.axis_index("c")
        base = wid * _B_PER_W
        pltpu.sync_copy(idx_hbm.at[pl.ds(base, _B_PER_W)], idx_v)
        pltpu.async_copy(tw_hbm.at[idx_v], rows_v, sem).wait()
        pltpu.sync_copy(rows_v, out_hbm.at[pl.ds(base, _B_PER_W)])

    return gather_kernel(tw, idxp)


def _mlp_body(hp_ref, idx_ref, bh_ref, wo_ref, bo_ref, out_ref, h_s):
    @pl.when(pl.program_id(0) == 0)
    def _():
        hi = (idx_ref[...] >= _HV).astype(jnp.float32)
        hp = hp_ref[:, : _EMBED] * (1.0 - hi) + hp_ref[:, _EMBED :] * hi
        h_s[...] = jnp.maximum(hp + bh_ref[...], 0.0)

    out = lax.dot_general(
        wo_ref[...], h_s[...], (((0,), (1,)), ((), ())),
        preferred_element_type=jnp.float32)
    ones = jnp.ones((1, _BATCH), dtype=jnp.float32)
    out_ref[...] = out + lax.dot_general(
        bo_ref[...], ones, (((0,), (0,)), ((), ())),
        preferred_element_type=jnp.float32)


def _tc_mlp(hp, idx, b_h, W_o, b_o):
    return pl.pallas_call(
        _mlp_body,
        grid=(_NV,),
        in_specs=[
            pl.BlockSpec((_BATCH, _RW), lambda i: (0, 0)),
            pl.BlockSpec((_BATCH, 1), lambda i: (0, 0)),
            pl.BlockSpec((1, _EMBED), lambda i: (0, 0)),
            pl.BlockSpec((_EMBED, _BN), lambda i: (0, i)),
            pl.BlockSpec((1, _BN), lambda i: (0, i)),
        ],
        out_specs=pl.BlockSpec((_BN, _BATCH), lambda i: (i, 0)),
        out_shape=jax.ShapeDtypeStruct((_VOCAB, _BATCH), jnp.float32),
        scratch_shapes=[pltpu.VMEM((_BATCH, _EMBED), jnp.float32)],
        compiler_params=pltpu.CompilerParams(
            dimension_semantics=("arbitrary",),
        ),
    )(hp, idx, b_h, W_o, b_o)


@jax.jit
def kernel(x, table, W_h, b_h, W_o, b_o):
    idx = x.astype(jnp.int32)
    table_t = jnp.transpose(table)
    tw = _tc_table_transform(table_t, W_h)
    idxp = jnp.where(idx >= _HV, idx - _HV, idx)
    hp = _sc_gather(tw, idxp)
    out_t = _tc_mlp(hp, jnp.reshape(idx, (_BATCH, 1)),
                    jnp.reshape(b_h, (1, _EMBED)), W_o,
                    jnp.reshape(b_o, (1, _VOCAB)))
    return jnp.transpose(out_t)


# speedup vs baseline: 1.2088x; 1.0140x over previous
import functools

import jax
import jax.numpy as jnp
from jax import lax
from jax.experimental import pallas as pl
from jax.experimental.pallas import tpu as pltpu
from jax.experimental.pallas import tpu_sc as plsc

_VOCAB = 100000
_EMBED = 64
_BATCH = 1024
_PK = 2
_RW = _PK * _EMBED
_HV = 51200

_NC = 2
_NS = 16
_NW = _NC * _NS
_B_PER_W = _BATCH // _NW
_L = 16

_BT = 12800
_NT = _HV // _BT
_BN = 4096
_NV = pl.cdiv(_VOCAB, _BN)


def _tw_body(tt_lo_ref, tt_hi_ref, wh_ref, out_ref):
    lo = lax.dot_general(
        tt_lo_ref[...], wh_ref[...], (((0,), (0,)), ((), ())),
        preferred_element_type=jnp.float32)
    hi = lax.dot_general(
        tt_hi_ref[...], wh_ref[...], (((0,), (0,)), ((), ())),
        preferred_element_type=jnp.float32)
    out_ref[...] = jnp.concatenate([lo, hi], axis=1)


def _tc_table_transform(table_t, W_h):
    return pl.pallas_call(
        _tw_body,
        grid=(_NT,),
        in_specs=[
            pl.BlockSpec((_EMBED, _BT), lambda i: (0, i)),
            pl.BlockSpec((_EMBED, _BT), lambda i: (0, i + _NT)),
            pl.BlockSpec((_EMBED, _EMBED), lambda i: (0, 0)),
        ],
        out_specs=pl.BlockSpec((_BT, _RW), lambda i: (i, 0)),
        out_shape=jax.ShapeDtypeStruct((_HV, _RW), jnp.float32),
        compiler_params=pltpu.CompilerParams(
            dimension_semantics=("arbitrary",),
        ),
    )(table_t, table_t, W_h)


def _sc_gather(tw, idxp):
    mesh = plsc.VectorSubcoreMesh(core_axis_name="c", subcore_axis_name="s")

    @functools.partial(
        pl.kernel,
        mesh=mesh,
        out_type=jax.ShapeDtypeStruct((_BATCH, _RW), jnp.float32),
        scratch_types=[
            pltpu.VMEM((_B_PER_W,), jnp.int32),
            pltpu.VMEM((_B_PER_W, _RW), jnp.float32),
            pltpu.SemaphoreType.DMA,
        ],
    )
    def gather_kernel(tw_hbm, idx_hbm, out_hbm, idx_v, rows_v, sem):
        wid = lax.axis_index("s") * _NC + lax.axis_index("c")
        base = wid * _B_PER_W
        pltpu.sync_copy(idx_hbm.at[pl.ds(base, _B_PER_W)], idx_v)
        pltpu.async_copy(tw_hbm.at[idx_v], rows_v, sem).wait()
        pltpu.sync_copy(rows_v, out_hbm.at[pl.ds(base, _B_PER_W)])

    return gather_kernel(tw, idxp)


def _mlp_body(hp_ref, idx_ref, bh_ref, wo_ref, bo_ref, out_ref, h_s):
    @pl.when(pl.program_id(0) == 0)
    def _():
        hi = (idx_ref[...] >= _HV).astype(jnp.float32)
        hp = hp_ref[:, : _EMBED] * (1.0 - hi) + hp_ref[:, _EMBED :] * hi
        h_s[...] = jnp.maximum(hp + bh_ref[...], 0.0)

    out = lax.dot_general(
        wo_ref[...], h_s[...], (((0,), (1,)), ((), ())),
        preferred_element_type=jnp.float32)
    ones = jnp.ones((1, _BATCH), dtype=jnp.float32)
    out_ref[...] = out + lax.dot_general(
        bo_ref[...], ones, (((0,), (0,)), ((), ())),
        preferred_element_type=jnp.float32)


def _tc_mlp(hp, idx, b_h, W_o, b_o):
    return pl.pallas_call(
        _mlp_body,
        grid=(_NV,),
        in_specs=[
            pl.BlockSpec((_BATCH, _RW), lambda i: (0, 0)),
            pl.BlockSpec((_BATCH, 1), lambda i: (0, 0)),
            pl.BlockSpec((1, _EMBED), lambda i: (0, 0)),
            pl.BlockSpec((_EMBED, _BN), lambda i: (0, i)),
            pl.BlockSpec((1, _BN), lambda i: (0, i)),
        ],
        out_specs=pl.BlockSpec((_BN, _BATCH), lambda i: (i, 0)),
        out_shape=jax.ShapeDtypeStruct((_VOCAB, _BATCH), jnp.float32),
        scratch_shapes=[pltpu.VMEM((_BATCH, _EMBED), jnp.float32)],
        compiler_params=pltpu.CompilerParams(
            dimension_semantics=("arbitrary",),
        ),
    )(hp, idx, b_h, W_o, b_o)


@jax.jit
def kernel(x, table, W_h, b_h, W_o, b_o):
    idx = x.astype(jnp.int32)
    table_t = jnp.transpose(table)
    tw = _tc_table_transform(table_t, W_h)
    idxp = jnp.where(idx >= _HV, idx - _HV, idx)
    hp = _sc_gather(tw, idxp)
    out_t = _tc_mlp(hp, jnp.reshape(idx, (_BATCH, 1)),
                    jnp.reshape(b_h, (1, _EMBED)), W_o,
                    jnp.reshape(b_o, (1, _VOCAB)))
    return jnp.transpose(out_t)


# speedup vs baseline: 1.2109x; 1.0017x over previous
import functools

import jax
import jax.numpy as jnp
from jax import lax
from jax.experimental import pallas as pl
from jax.experimental.pallas import tpu as pltpu
from jax.experimental.pallas import tpu_sc as plsc

_VOCAB = 100000
_EMBED = 64
_BATCH = 1024
_PK = 2
_RW = _PK * _EMBED
_HV = 51200

_NC = 2
_NS = 16
_NW = _NC * _NS
_B_PER_W = _BATCH // _NW
_L = 16

_BT = 6400
_NT = _HV // _BT
_BN = 4096
_NV = pl.cdiv(_VOCAB, _BN)


def _tw_body(tt_lo_ref, tt_hi_ref, wh_ref, out_ref):
    lo = lax.dot_general(
        tt_lo_ref[...], wh_ref[...], (((0,), (0,)), ((), ())),
        preferred_element_type=jnp.float32)
    hi = lax.dot_general(
        tt_hi_ref[...], wh_ref[...], (((0,), (0,)), ((), ())),
        preferred_element_type=jnp.float32)
    out_ref[...] = jnp.concatenate([lo, hi], axis=1)


def _tc_table_transform(table_t, W_h):
    return pl.pallas_call(
        _tw_body,
        grid=(_NT,),
        in_specs=[
            pl.BlockSpec((_EMBED, _BT), lambda i: (0, i)),
            pl.BlockSpec((_EMBED, _BT), lambda i: (0, i + _NT)),
            pl.BlockSpec((_EMBED, _EMBED), lambda i: (0, 0)),
        ],
        out_specs=pl.BlockSpec((_BT, _RW), lambda i: (i, 0)),
        out_shape=jax.ShapeDtypeStruct((_HV, _RW), jnp.float32),
        compiler_params=pltpu.CompilerParams(
            dimension_semantics=("arbitrary",),
        ),
    )(table_t, table_t, W_h)


def _sc_gather(tw, idxp):
    mesh = plsc.VectorSubcoreMesh(core_axis_name="c", subcore_axis_name="s")

    @functools.partial(
        pl.kernel,
        mesh=mesh,
        out_type=jax.ShapeDtypeStruct((_BATCH, _RW), jnp.float32),
        scratch_types=[
            pltpu.VMEM((_B_PER_W,), jnp.int32),
            pltpu.VMEM((_B_PER_W, _RW), jnp.float32),
            pltpu.SemaphoreType.DMA,
        ],
    )
    def gather_kernel(tw_hbm, idx_hbm, out_hbm, idx_v, rows_v, sem):
        wid = lax.axis_index("s") * _NC + lax.axis_index("c")
        base = wid * _B_PER_W
        pltpu.sync_copy(idx_hbm.at[pl.ds(base, _B_PER_W)], idx_v)
        pltpu.async_copy(tw_hbm.at[idx_v], rows_v, sem).wait()
        pltpu.sync_copy(rows_v, out_hbm.at[pl.ds(base, _B_PER_W)])

    return gather_kernel(tw, idxp)


def _mlp_body(hp_ref, idx_ref, bh_ref, wo_ref, bo_ref, out_ref, h_s):
    @pl.when(pl.program_id(0) == 0)
    def _():
        hi = (idx_ref[...] >= _HV).astype(jnp.float32)
        hp = hp_ref[:, : _EMBED] * (1.0 - hi) + hp_ref[:, _EMBED :] * hi
        h_s[...] = jnp.maximum(hp + bh_ref[...], 0.0)

    out = lax.dot_general(
        wo_ref[...], h_s[...], (((0,), (1,)), ((), ())),
        preferred_element_type=jnp.float32)
    ones = jnp.ones((1, _BATCH), dtype=jnp.float32)
    out_ref[...] = out + lax.dot_general(
        bo_ref[...], ones, (((0,), (0,)), ((), ())),
        preferred_element_type=jnp.float32)


def _tc_mlp(hp, idx, b_h, W_o, b_o):
    return pl.pallas_call(
        _mlp_body,
        grid=(_NV,),
        in_specs=[
            pl.BlockSpec((_BATCH, _RW), lambda i: (0, 0)),
            pl.BlockSpec((_BATCH, 1), lambda i: (0, 0)),
            pl.BlockSpec((1, _EMBED), lambda i: (0, 0)),
            pl.BlockSpec((_EMBED, _BN), lambda i: (0, i)),
            pl.BlockSpec((1, _BN), lambda i: (0, i)),
        ],
        out_specs=pl.BlockSpec((_BN, _BATCH), lambda i: (i, 0)),
        out_shape=jax.ShapeDtypeStruct((_VOCAB, _BATCH), jnp.float32),
        scratch_shapes=[pltpu.VMEM((_BATCH, _EMBED), jnp.float32)],
        compiler_params=pltpu.CompilerParams(
            dimension_semantics=("arbitrary",),
        ),
    )(hp, idx, b_h, W_o, b_o)


@jax.jit
def kernel(x, table, W_h, b_h, W_o, b_o):
    idx = x.astype(jnp.int32)
    table_t = jnp.transpose(table)
    tw = _tc_table_transform(table_t, W_h)
    idxp = jnp.where(idx >= _HV, idx - _HV, idx)
    hp = _sc_gather(tw, idxp)
    out_t = _tc_mlp(hp, jnp.reshape(idx, (_BATCH, 1)),
                    jnp.reshape(b_h, (1, _EMBED)), W_o,
                    jnp.reshape(b_o, (1, _VOCAB)))
    return jnp.transpose(out_t)
